# baseline (speedup 1.0000x reference)
import functools

import jax
import jax.numpy as jnp
from jax import lax
from jax.experimental import pallas as pl
from jax.experimental.pallas import tpu as pltpu
from jax.experimental.pallas import tpu_sc as plsc

BATCH = 16384
TOPIC_DIM = 128
SUBTOPIC_DIM = 64
OUT_DIM = TOPIC_DIM + SUBTOPIC_DIM
CHUNK = 128


@functools.cache
def _build():
    info = plsc.get_sparse_core_info()
    num_workers = info.num_cores * info.num_subcores
    rows_per_worker = BATCH // num_workers
    n_chunks = rows_per_worker // CHUNK
    mesh = plsc.VectorSubcoreMesh(core_axis_name="c", subcore_axis_name="s")

    scratch = [
        pltpu.VMEM((n_chunks, CHUNK), jnp.int32),
        pltpu.VMEM((n_chunks, CHUNK), jnp.int32),
        pltpu.VMEM((rows_per_worker, TOPIC_DIM), jnp.float32),
        pltpu.VMEM((rows_per_worker, SUBTOPIC_DIM), jnp.float32),
        pltpu.SemaphoreType.DMA,
        pltpu.SemaphoreType.DMA,
        pltpu.SemaphoreType.DMA,
    ]

    @functools.partial(
        pl.kernel,
        mesh=mesh,
        out_type=jax.ShapeDtypeStruct((BATCH, OUT_DIM), jnp.float32),
        scratch_types=scratch,
        compiler_params=pltpu.CompilerParams(use_tc_tiling_on_sc=False),
    )
    def enc(t_idx_hbm, s_idx_hbm, t_tab_hbm, s_tab_hbm, out_hbm,
            t_idx_v, s_idx_v, t_rows, s_rows, t_sem, s_sem, w_sem):
        wid = lax.axis_index("s") * info.num_cores + lax.axis_index("c")
        base = wid * rows_per_worker
        idx_row0 = wid * n_chunks

        ti_cp = pltpu.async_copy(
            t_idx_hbm.at[pl.ds(idx_row0, n_chunks)], t_idx_v, t_sem)
        si_cp = pltpu.async_copy(
            s_idx_hbm.at[pl.ds(idx_row0, n_chunks)], s_idx_v, s_sem)

        t_cps = []
        s_cps = []
        ti_cp.wait()
        si_cp.wait()
        for c in range(n_chunks):
            rows = pl.ds(c * CHUNK, CHUNK)
            t_cps.append(pltpu.async_copy(
                t_tab_hbm.at[t_idx_v.at[c]], t_rows.at[rows], t_sem))
            s_cps.append(pltpu.async_copy(
                s_tab_hbm.at[s_idx_v.at[c]], s_rows.at[rows], s_sem))

        half = rows_per_worker // 2
        w_cps = []
        for h in range(2):
            rows_h = pl.ds(h * half, half)
            out_rows = pl.ds(base + h * half, half)
            for cp in t_cps[2 * h:2 * h + 2]:
                cp.wait()
            w_cps.append(pltpu.async_copy(
                t_rows.at[rows_h],
                out_hbm.at[out_rows, pl.ds(0, TOPIC_DIM)], w_sem))
            for cp in s_cps[2 * h:2 * h + 2]:
                cp.wait()
            w_cps.append(pltpu.async_copy(
                s_rows.at[rows_h],
                out_hbm.at[out_rows, pl.ds(TOPIC_DIM, SUBTOPIC_DIM)], w_sem))
        for cp in w_cps:
            cp.wait()

    return enc


def kernel(topic, subtopic, topic_table, subtopic_table):
    enc = _build()
    n_rows = BATCH // CHUNK
    return enc(topic.astype(jnp.int32).reshape(n_rows, CHUNK),
               subtopic.astype(jnp.int32).reshape(n_rows, CHUNK),
               topic_table, subtopic_table)
